# baseline (speedup 1.0000x reference)
import functools

import jax
import jax.numpy as jnp
from jax import lax
from jax.experimental import pallas as pl
from jax.experimental.pallas import tpu as pltpu
from jax.experimental.pallas import tpu_sc as plsc

ALPHA = 1.0
BETA = 1.0
LAMB = 1e-4
TAU = 0.1
DELTA = 1e-4

BIG = 3.0e38
BIGI = 2**30
NS_ITERS = 30
TOPW = 16
GW = 128


CHUNK = 200


def _knn_stats_kernel(q_ref, w_ref, b_ref, x_ref,
                      t_ref, s_ref, cs_ref, topv_ref, topi_ref, sc_scr,
                      *, blk, k, nh):
    h = pl.program_id(0)
    i = pl.program_id(1)

    @pl.when(i == 0)
    def _init():
        t_ref[0] = (
            jnp.dot(q_ref[...], w_ref[...], preferred_element_type=jnp.float32)
            + b_ref[...]
        )
        s_ref[0] = jnp.zeros_like(s_ref[0])
        cs_ref[0] = jnp.zeros_like(cs_ref[0])
        topv_ref[0] = jnp.full(topv_ref.shape[1:], BIG, jnp.float32)
        topi_ref[0] = jnp.zeros(topi_ref.shape[1:], jnp.int32)

    x = x_ref[...]
    t = t_ref[0]
    cs_ref[0] += jnp.sum(x, axis=0, keepdims=True)
    s_ref[0] += lax.dot_general(x, x, (((0,), (0,)), ((), ())),
                                preferred_element_type=jnp.float32)

    rn = jnp.sum(x * x, axis=1)
    sc_scr[...] = rn[:, None] - 2.0 * lax.dot_general(
        x, t, (((1,), (1,)), ((), ())), preferred_element_type=jnp.float32)

    b_count = t.shape[0]
    nrun = topv_ref.shape[1]
    base = (h * nh + i) * blk

    def merge_chunk(c, carry):
        ch = sc_scr[pl.ds(c * CHUNK, CHUNK), :]
        theta = topv_ref[0, k - 1:k, :]
        cnt = jnp.sum((ch < theta).astype(jnp.int32), axis=0, keepdims=True)
        n_hit = jnp.minimum(jnp.max(cnt), k)

        @pl.when(n_hit > 0)
        def _merge():
            gci = (lax.broadcasted_iota(jnp.int32, ch.shape, 0)
                   + (base + c * CHUNK))
            rows16 = lax.broadcasted_iota(jnp.int32, (nrun, b_count), 0)

            def extract(j, carry2):
                chv, newv, newi = carry2
                m = jnp.min(chv, axis=0, keepdims=True)
                ix = jnp.min(jnp.where(chv == m, gci, BIGI),
                             axis=0, keepdims=True)
                chv = jnp.where((chv == m) & (gci == ix), BIG, chv)
                newv = jnp.where(rows16 == j, m, newv)
                newi = jnp.where(rows16 == j, ix, newi)
                return chv, newv, newi

            newv0 = jnp.full((nrun, b_count), BIG, jnp.float32)
            newi0 = jnp.zeros((nrun, b_count), jnp.int32)
            _, newv, newi = lax.fori_loop(0, n_hit, extract,
                                          (ch, newv0, newi0))

            cv = jnp.concatenate([topv_ref[0], newv], axis=0)
            ci = jnp.concatenate([topi_ref[0], newi], axis=0)
            rows = lax.broadcasted_iota(jnp.int32, cv.shape, 0)
            nv, ni = [], []
            for _ in range(k):
                m = jnp.min(cv, axis=0, keepdims=True)
                r = jnp.min(jnp.where(cv == m, rows, BIGI),
                            axis=0, keepdims=True)
                hit = rows == r
                ix = jnp.sum(jnp.where(hit, ci, 0), axis=0, keepdims=True)
                nv.append(m)
                ni.append(ix)
                cv = jnp.where(hit, BIG, cv)
            nv.append(jnp.full((nrun - k, b_count), BIG, jnp.float32))
            ni.append(jnp.zeros((nrun - k, b_count), jnp.int32))
            topv_ref[0] = jnp.concatenate(nv, axis=0)
            topi_ref[0] = jnp.concatenate(ni, axis=0)

        return carry

    lax.fori_loop(0, blk // CHUNK, merge_chunk, 0)


def _sc_gather_kernel(qi_hbm, pi_hbm, pw_hbm, oi_hbm, ow_hbm,
                      idx_v, rows_i, rows_w, sem_i, sem_w, *,
                      n_workers, rows_per, num_cores):
    wid = lax.axis_index("s") * num_cores + lax.axis_index("c")

    @pl.when(wid < n_workers)
    def _():
        base = wid * rows_per
        pltpu.sync_copy(qi_hbm.at[pl.ds(base, rows_per)], idx_v)
        pltpu.async_copy(pi_hbm.at[idx_v], rows_i, sem_i).wait()
        pltpu.async_copy(pw_hbm.at[idx_v], rows_w, sem_w).wait()
        pltpu.sync_copy(rows_i, oi_hbm.at[pl.ds(base, rows_per)])
        pltpu.sync_copy(rows_w, ow_hbm.at[pl.ds(base, rows_per)])


def _final_kernel(t_ref, s_ref, cs_ref, topv_ref, topi_ref,
                  gpi_ref, gpw_ref, w_ref, b_ref, out_ref, *, n_db, k):
    d = s_ref.shape[1]
    b_count = t_ref.shape[1]
    rr = lax.broadcasted_iota(jnp.int32, (d, d), 0)
    cc = lax.broadcasted_iota(jnp.int32, (d, d), 1)
    eye = (rr == cc).astype(jnp.float32)

    mu_x = (cs_ref[0] + cs_ref[1]) / n_db
    cov_x = ((s_ref[0] + s_ref[1]) / n_db
             - lax.dot_general(mu_x, mu_x, (((0,), (0,)), ((), ())),
                               preferred_element_type=jnp.float32)
             + DELTA * eye)
    t = t_ref[0]
    mu_t = jnp.sum(t, axis=0, keepdims=True) / b_count
    tc = t - mu_t
    cov_t = (lax.dot_general(tc, tc, (((0,), (0,)), ((), ())),
                             preferred_element_type=jnp.float32) / b_count
             + DELTA * eye)
    loss_mean = jnp.sum((mu_t - mu_x) ** 2)

    def ns_sqrt(a):
        c = jnp.sum(a * eye)
        y = a / c
        z = eye
        for _ in range(NS_ITERS):
            tm = 1.5 * eye - 0.5 * jnp.dot(z, y, preferred_element_type=jnp.float32)
            y = jnp.dot(y, tm, preferred_element_type=jnp.float32)
            z = jnp.dot(tm, z, preferred_element_type=jnp.float32)
        return y * jnp.sqrt(c)

    st = ns_sqrt(cov_t)
    m_mid = jnp.dot(jnp.dot(st, cov_x, preferred_element_type=jnp.float32), st,
                    preferred_element_type=jnp.float32)
    sq = ns_sqrt(m_mid)
    loss_cov = jnp.sum(cov_x * eye) + jnp.sum(cov_t * eye) - 2.0 * jnp.sum(sq * eye)
    loss_dist = jnp.maximum(loss_mean + loss_cov, 0.0)

    cv = jnp.concatenate([topv_ref[0], topv_ref[1]], axis=0)
    ci = jnp.concatenate([topi_ref[0], topi_ref[1]], axis=0)
    rows = lax.broadcasted_iota(jnp.int32, cv.shape, 0)
    mv, mi = [], []
    for _ in range(k):
        m = jnp.min(cv, axis=0, keepdims=True)
        r = jnp.min(jnp.where(cv == m, rows, BIGI), axis=0, keepdims=True)
        hit = rows == r
        ix = jnp.sum(jnp.where(hit, ci, 0), axis=0, keepdims=True)
        mv.append(m)
        mi.append(ix)
        cv = jnp.where(hit, BIG, cv)
    mv = jnp.concatenate(mv, axis=0)
    mi = jnp.concatenate(mi, axis=0)

    rb = lax.broadcasted_iota(jnp.int32, (b_count, b_count), 0)
    cb = lax.broadcasted_iota(jnp.int32, (b_count, b_count), 1)
    eye_b = (rb == cb).astype(jnp.float32)
    pv = lax.dot_general(eye_b, mv, (((1,), (1,)), ((), ())),
                         preferred_element_type=jnp.float32)
    post_i = lax.dot_general(eye_b, mi.astype(jnp.float32),
                             (((1,), (1,)), ((), ())),
                             preferred_element_type=jnp.float32).astype(jnp.int32)

    mrow = jnp.min(pv, axis=1, keepdims=True)
    ew = jnp.exp(-(pv - mrow) / TAU)
    post_w = ew / jnp.sum(ew, axis=1, keepdims=True)

    pre_i = gpi_ref[:, :k]
    pre_w = gpw_ref[:, :k]
    all_idx = jnp.concatenate([pre_i, post_i], axis=1)
    a3 = all_idx[:, :, None]
    eq_tt = a3 == all_idx[:, None, :]
    it_t = lax.broadcasted_iota(jnp.int32, eq_tt.shape, 1)
    it_s = lax.broadcasted_iota(jnp.int32, eq_tt.shape, 2)
    first = ~jnp.any(eq_tt & (it_s < it_t), axis=2)
    p_m = jnp.sum((a3 == pre_i[:, None, :]).astype(jnp.float32)
                  * pre_w[:, None, :], axis=2)
    q_m = jnp.sum((a3 == post_i[:, None, :]).astype(jnp.float32)
                  * post_w[:, None, :], axis=2)
    p_r = jnp.where(first, jnp.maximum(p_m, 1e-8), 0.0)
    p_h = p_r / jnp.sum(p_r, axis=1, keepdims=True)
    q_r = jnp.where(first, jnp.maximum(q_m, 1e-8), 0.0)
    q_h = q_r / jnp.sum(q_r, axis=1, keepdims=True)
    lp = jnp.log(jnp.where(first, p_h, 1.0))
    lq = jnp.log(jnp.where(first, q_h, 1.0))
    loss_knn = jnp.sum(jnp.where(first, p_h * (lp - lq), 0.0)) / b_count

    loss_reg = (jnp.sum(w_ref[...] ** 2) + jnp.sum(b_ref[...] ** 2)) / 2.0
    total = ALPHA * loss_dist + BETA * loss_knn + LAMB * loss_reg

    ji = lax.broadcasted_iota(jnp.int32, out_ref.shape, 1)
    out_ref[...] = (jnp.where(ji == 0, total, 0.0)
                    + jnp.where(ji == 1, loss_dist, 0.0)
                    + jnp.where(ji == 2, loss_knn, 0.0))


def kernel(q_batch, q_indices, X, W, b, pre_weights, pre_indices):
    n_db, d = X.shape
    b_count = q_batch.shape[0]
    k = pre_indices.shape[1]
    blk = 2000
    nsteps = n_db // blk
    b2 = b.reshape(1, d).astype(jnp.float32)

    nh = nsteps // 2
    t_q, s_acc, cs, topv, topi = pl.pallas_call(
        functools.partial(_knn_stats_kernel, blk=blk, k=k, nh=nh),
        grid=(2, nh),
        in_specs=[
            pl.BlockSpec((b_count, d), lambda h, i: (0, 0)),
            pl.BlockSpec((d, d), lambda h, i: (0, 0)),
            pl.BlockSpec((1, d), lambda h, i: (0, 0)),
            pl.BlockSpec((blk, d), lambda h, i, nh=nh: (h * nh + i, 0)),
        ],
        out_specs=[
            pl.BlockSpec((1, b_count, d), lambda h, i: (h, 0, 0)),
            pl.BlockSpec((1, d, d), lambda h, i: (h, 0, 0)),
            pl.BlockSpec((1, 1, d), lambda h, i: (h, 0, 0)),
            pl.BlockSpec((1, TOPW, b_count), lambda h, i: (h, 0, 0)),
            pl.BlockSpec((1, TOPW, b_count), lambda h, i: (h, 0, 0)),
        ],
        out_shape=[
            jax.ShapeDtypeStruct((2, b_count, d), jnp.float32),
            jax.ShapeDtypeStruct((2, d, d), jnp.float32),
            jax.ShapeDtypeStruct((2, 1, d), jnp.float32),
            jax.ShapeDtypeStruct((2, TOPW, b_count), jnp.float32),
            jax.ShapeDtypeStruct((2, TOPW, b_count), jnp.int32),
        ],
        scratch_shapes=[pltpu.VMEM((blk, b_count), jnp.float32)],
        compiler_params=pltpu.CompilerParams(
            dimension_semantics=("parallel", "arbitrary")),
    )(q_batch.astype(jnp.float32), W.astype(jnp.float32), b2,
      X.astype(jnp.float32))

    info = plsc.get_sparse_core_info()
    num_cores = info.num_cores
    n_workers = 16
    rows_per = b_count // n_workers
    qi = q_indices.astype(jnp.int32)
    pi_pad = jnp.pad(pre_indices.astype(jnp.int32), ((0, 0), (0, GW - k)))
    pw_pad = jnp.pad(pre_weights.astype(jnp.float32), ((0, 0), (0, GW - k)))
    mesh = plsc.VectorSubcoreMesh(core_axis_name="c", subcore_axis_name="s")
    gpi, gpw = pl.kernel(
        functools.partial(_sc_gather_kernel, n_workers=n_workers,
                          rows_per=rows_per, num_cores=num_cores),
        out_type=[
            jax.ShapeDtypeStruct((b_count, GW), jnp.int32),
            jax.ShapeDtypeStruct((b_count, GW), jnp.float32),
        ],
        mesh=mesh,
        scratch_types=[
            pltpu.VMEM((rows_per,), jnp.int32),
            pltpu.VMEM((rows_per, GW), jnp.int32),
            pltpu.VMEM((rows_per, GW), jnp.float32),
            pltpu.SemaphoreType.DMA,
            pltpu.SemaphoreType.DMA,
        ],
    )(qi, pi_pad, pw_pad)

    out = pl.pallas_call(
        functools.partial(_final_kernel, n_db=n_db, k=k),
        in_specs=[pl.BlockSpec(a.shape, lambda n=a.ndim: (0,) * n)
                  for a in (t_q, s_acc, cs, topv, topi, gpi, gpw)]
        + [pl.BlockSpec((d, d), lambda: (0, 0)),
           pl.BlockSpec((1, d), lambda: (0, 0))],
        out_specs=pl.BlockSpec((1, 8), lambda: (0, 0)),
        out_shape=jax.ShapeDtypeStruct((1, 8), jnp.float32),
    )(t_q, s_acc, cs, topv, topi, gpi, gpw, W.astype(jnp.float32), b2)

    return (out[0, 0], out[0, 1], out[0, 2])


# speedup vs baseline: 1.0432x; 1.0432x over previous
import functools

import jax
import jax.numpy as jnp
from jax import lax
from jax.experimental import pallas as pl
from jax.experimental.pallas import tpu as pltpu
from jax.experimental.pallas import tpu_sc as plsc

ALPHA = 1.0
BETA = 1.0
LAMB = 1e-4
TAU = 0.1
DELTA = 1e-4

BIG = 3.0e38
BIGI = 2**30
NS_ITERS = 30
TOPW = 16
GW = 128


CHUNK = 200


def _knn_stats_kernel(q_ref, w_ref, b_ref, x_ref,
                      t_ref, s_ref, cs_ref, topv_ref, topi_ref, sc_scr,
                      *, blk, k):
    i = pl.program_id(0)

    @pl.when(i == 0)
    def _init():
        t_ref[...] = (
            jnp.dot(q_ref[...], w_ref[...], preferred_element_type=jnp.float32)
            + b_ref[...]
        )
        s_ref[...] = jnp.zeros_like(s_ref)
        cs_ref[...] = jnp.zeros_like(cs_ref)
        topv_ref[...] = jnp.full(topv_ref.shape, BIG, jnp.float32)
        topi_ref[...] = jnp.zeros(topi_ref.shape, jnp.int32)

    x = x_ref[...]
    t = t_ref[...]
    cs_ref[...] += jnp.sum(x, axis=0, keepdims=True)
    s_ref[...] += lax.dot_general(x, x, (((0,), (0,)), ((), ())),
                                  preferred_element_type=jnp.float32)

    rn = jnp.sum(x * x, axis=1)
    sc_scr[...] = rn[:, None] - 2.0 * lax.dot_general(
        x, t, (((1,), (1,)), ((), ())), preferred_element_type=jnp.float32)

    b_count = t.shape[0]
    nrun = topv_ref.shape[0]

    def merge_chunk(c, carry):
        ch = sc_scr[pl.ds(c * CHUNK, CHUNK), :]
        theta = topv_ref[k - 1:k, :]
        cnt = jnp.sum((ch < theta).astype(jnp.int32), axis=0, keepdims=True)
        n_hit = jnp.minimum(jnp.max(cnt), k)

        @pl.when(n_hit > 0)
        def _merge():
            gci = (lax.broadcasted_iota(jnp.int32, ch.shape, 0)
                   + (i * blk + c * CHUNK))
            rows16 = lax.broadcasted_iota(jnp.int32, (nrun, b_count), 0)

            def extract(j, carry2):
                chv, newv, newi = carry2
                m = jnp.min(chv, axis=0, keepdims=True)
                ix = jnp.min(jnp.where(chv == m, gci, BIGI),
                             axis=0, keepdims=True)
                chv = jnp.where((chv == m) & (gci == ix), BIG, chv)
                newv = jnp.where(rows16 == j, m, newv)
                newi = jnp.where(rows16 == j, ix, newi)
                return chv, newv, newi

            newv0 = jnp.full((nrun, b_count), BIG, jnp.float32)
            newi0 = jnp.zeros((nrun, b_count), jnp.int32)
            _, newv, newi = lax.fori_loop(0, n_hit, extract,
                                          (ch, newv0, newi0))

            cv = jnp.concatenate([topv_ref[...], newv], axis=0)
            ci = jnp.concatenate([topi_ref[...], newi], axis=0)
            rows = lax.broadcasted_iota(jnp.int32, cv.shape, 0)
            nv, ni = [], []
            for _ in range(k):
                m = jnp.min(cv, axis=0, keepdims=True)
                r = jnp.min(jnp.where(cv == m, rows, BIGI),
                            axis=0, keepdims=True)
                hit = rows == r
                ix = jnp.sum(jnp.where(hit, ci, 0), axis=0, keepdims=True)
                nv.append(m)
                ni.append(ix)
                cv = jnp.where(hit, BIG, cv)
            nv.append(jnp.full((nrun - k, b_count), BIG, jnp.float32))
            ni.append(jnp.zeros((nrun - k, b_count), jnp.int32))
            topv_ref[...] = jnp.concatenate(nv, axis=0)
            topi_ref[...] = jnp.concatenate(ni, axis=0)

        return carry

    lax.fori_loop(0, blk // CHUNK, merge_chunk, 0)


def _sc_gather_kernel(qi_hbm, pi_hbm, pw_hbm, oi_hbm, ow_hbm,
                      idx_v, rows_i, rows_w, sem_i, sem_w, *,
                      n_workers, rows_per, num_cores):
    wid = lax.axis_index("s") * num_cores + lax.axis_index("c")

    @pl.when(wid < n_workers)
    def _():
        base = wid * rows_per
        pltpu.sync_copy(qi_hbm.at[pl.ds(base, rows_per)], idx_v)
        pltpu.async_copy(pi_hbm.at[idx_v], rows_i, sem_i).wait()
        pltpu.async_copy(pw_hbm.at[idx_v], rows_w, sem_w).wait()
        pltpu.sync_copy(rows_i, oi_hbm.at[pl.ds(base, rows_per)])
        pltpu.sync_copy(rows_w, ow_hbm.at[pl.ds(base, rows_per)])


def _final_kernel(t_ref, s_ref, cs_ref, topv_ref, topi_ref,
                  gpi_ref, gpw_ref, w_ref, b_ref, out_ref, *, n_db, k):
    d = s_ref.shape[0]
    b_count = t_ref.shape[0]
    rr = lax.broadcasted_iota(jnp.int32, (d, d), 0)
    cc = lax.broadcasted_iota(jnp.int32, (d, d), 1)
    eye = (rr == cc).astype(jnp.float32)

    mu_x = cs_ref[...] / n_db
    cov_x = (s_ref[...] / n_db
             - lax.dot_general(mu_x, mu_x, (((0,), (0,)), ((), ())),
                               preferred_element_type=jnp.float32)
             + DELTA * eye)
    t = t_ref[...]
    mu_t = jnp.sum(t, axis=0, keepdims=True) / b_count
    tc = t - mu_t
    cov_t = (lax.dot_general(tc, tc, (((0,), (0,)), ((), ())),
                             preferred_element_type=jnp.float32) / b_count
             + DELTA * eye)
    loss_mean = jnp.sum((mu_t - mu_x) ** 2)

    def ns_sqrt(a):
        c = jnp.sum(a * eye)
        y = a / c
        z = eye
        for _ in range(NS_ITERS):
            tm = 1.5 * eye - 0.5 * jnp.dot(z, y, preferred_element_type=jnp.float32)
            y = jnp.dot(y, tm, preferred_element_type=jnp.float32)
            z = jnp.dot(tm, z, preferred_element_type=jnp.float32)
        return y * jnp.sqrt(c)

    st = ns_sqrt(cov_t)
    m_mid = jnp.dot(jnp.dot(st, cov_x, preferred_element_type=jnp.float32), st,
                    preferred_element_type=jnp.float32)
    sq = ns_sqrt(m_mid)
    loss_cov = jnp.sum(cov_x * eye) + jnp.sum(cov_t * eye) - 2.0 * jnp.sum(sq * eye)
    loss_dist = jnp.maximum(loss_mean + loss_cov, 0.0)

    rb = lax.broadcasted_iota(jnp.int32, (b_count, b_count), 0)
    cb = lax.broadcasted_iota(jnp.int32, (b_count, b_count), 1)
    eye_b = (rb == cb).astype(jnp.float32)
    topv_t = lax.dot_general(eye_b, topv_ref[...], (((1,), (1,)), ((), ())),
                             preferred_element_type=jnp.float32)
    topi_t = lax.dot_general(eye_b, topi_ref[...].astype(jnp.float32),
                             (((1,), (1,)), ((), ())),
                             preferred_element_type=jnp.float32).astype(jnp.int32)

    pv = topv_t[:, :k]
    post_i = topi_t[:, :k]
    mrow = jnp.min(pv, axis=1, keepdims=True)
    ew = jnp.exp(-(pv - mrow) / TAU)
    post_w = ew / jnp.sum(ew, axis=1, keepdims=True)

    pre_i = gpi_ref[:, :k]
    pre_w = gpw_ref[:, :k]
    all_idx = jnp.concatenate([pre_i, post_i], axis=1)
    a3 = all_idx[:, :, None]
    eq_tt = a3 == all_idx[:, None, :]
    it_t = lax.broadcasted_iota(jnp.int32, eq_tt.shape, 1)
    it_s = lax.broadcasted_iota(jnp.int32, eq_tt.shape, 2)
    first = ~jnp.any(eq_tt & (it_s < it_t), axis=2)
    p_m = jnp.sum((a3 == pre_i[:, None, :]).astype(jnp.float32)
                  * pre_w[:, None, :], axis=2)
    q_m = jnp.sum((a3 == post_i[:, None, :]).astype(jnp.float32)
                  * post_w[:, None, :], axis=2)
    p_r = jnp.where(first, jnp.maximum(p_m, 1e-8), 0.0)
    p_h = p_r / jnp.sum(p_r, axis=1, keepdims=True)
    q_r = jnp.where(first, jnp.maximum(q_m, 1e-8), 0.0)
    q_h = q_r / jnp.sum(q_r, axis=1, keepdims=True)
    lp = jnp.log(jnp.where(first, p_h, 1.0))
    lq = jnp.log(jnp.where(first, q_h, 1.0))
    loss_knn = jnp.sum(jnp.where(first, p_h * (lp - lq), 0.0)) / b_count

    loss_reg = (jnp.sum(w_ref[...] ** 2) + jnp.sum(b_ref[...] ** 2)) / 2.0
    total = ALPHA * loss_dist + BETA * loss_knn + LAMB * loss_reg

    ji = lax.broadcasted_iota(jnp.int32, out_ref.shape, 1)
    out_ref[...] = (jnp.where(ji == 0, total, 0.0)
                    + jnp.where(ji == 1, loss_dist, 0.0)
                    + jnp.where(ji == 2, loss_knn, 0.0))


def kernel(q_batch, q_indices, X, W, b, pre_weights, pre_indices):
    n_db, d = X.shape
    b_count = q_batch.shape[0]
    k = pre_indices.shape[1]
    blk = 2000
    nsteps = n_db // blk
    b2 = b.reshape(1, d).astype(jnp.float32)

    t_q, s_acc, cs, topv, topi = pl.pallas_call(
        functools.partial(_knn_stats_kernel, blk=blk, k=k),
        grid=(nsteps,),
        in_specs=[
            pl.BlockSpec((b_count, d), lambda i: (0, 0)),
            pl.BlockSpec((d, d), lambda i: (0, 0)),
            pl.BlockSpec((1, d), lambda i: (0, 0)),
            pl.BlockSpec((blk, d), lambda i: (i, 0)),
        ],
        out_specs=[
            pl.BlockSpec((b_count, d), lambda i: (0, 0)),
            pl.BlockSpec((d, d), lambda i: (0, 0)),
            pl.BlockSpec((1, d), lambda i: (0, 0)),
            pl.BlockSpec((TOPW, b_count), lambda i: (0, 0)),
            pl.BlockSpec((TOPW, b_count), lambda i: (0, 0)),
        ],
        out_shape=[
            jax.ShapeDtypeStruct((b_count, d), jnp.float32),
            jax.ShapeDtypeStruct((d, d), jnp.float32),
            jax.ShapeDtypeStruct((1, d), jnp.float32),
            jax.ShapeDtypeStruct((TOPW, b_count), jnp.float32),
            jax.ShapeDtypeStruct((TOPW, b_count), jnp.int32),
        ],
        scratch_shapes=[pltpu.VMEM((blk, b_count), jnp.float32)],
    )(q_batch.astype(jnp.float32), W.astype(jnp.float32), b2,
      X.astype(jnp.float32))

    info = plsc.get_sparse_core_info()
    num_cores = info.num_cores
    n_workers = 16
    rows_per = b_count // n_workers
    qi = q_indices.astype(jnp.int32)
    pi_pad = jnp.pad(pre_indices.astype(jnp.int32), ((0, 0), (0, GW - k)))
    pw_pad = jnp.pad(pre_weights.astype(jnp.float32), ((0, 0), (0, GW - k)))
    mesh = plsc.VectorSubcoreMesh(core_axis_name="c", subcore_axis_name="s")
    gpi, gpw = pl.kernel(
        functools.partial(_sc_gather_kernel, n_workers=n_workers,
                          rows_per=rows_per, num_cores=num_cores),
        out_type=[
            jax.ShapeDtypeStruct((b_count, GW), jnp.int32),
            jax.ShapeDtypeStruct((b_count, GW), jnp.float32),
        ],
        mesh=mesh,
        scratch_types=[
            pltpu.VMEM((rows_per,), jnp.int32),
            pltpu.VMEM((rows_per, GW), jnp.int32),
            pltpu.VMEM((rows_per, GW), jnp.float32),
            pltpu.SemaphoreType.DMA,
            pltpu.SemaphoreType.DMA,
        ],
    )(qi, pi_pad, pw_pad)

    out = pl.pallas_call(
        functools.partial(_final_kernel, n_db=n_db, k=k),
        in_specs=[pl.BlockSpec(a.shape, lambda: (0,) * a.ndim)
                  for a in (t_q, s_acc, cs, topv, topi, gpi, gpw)]
        + [pl.BlockSpec((d, d), lambda: (0, 0)),
           pl.BlockSpec((1, d), lambda: (0, 0))],
        out_specs=pl.BlockSpec((1, 8), lambda: (0, 0)),
        out_shape=jax.ShapeDtypeStruct((1, 8), jnp.float32),
    )(t_q, s_acc, cs, topv, topi, gpi, gpw, W.astype(jnp.float32), b2)

    return (out[0, 0], out[0, 1], out[0, 2])
